# baseline (speedup 1.0000x reference)
import functools

import jax
import jax.numpy as jnp
from jax import lax
from jax.experimental import pallas as pl
from jax.experimental.pallas import tpu as pltpu
from jax.experimental.pallas import tpu_sc as plsc

B, S = 4096, 200
D = 64
NW = 32
G = 8


def _make_kernel(nb):
    seq_per_w = nb // NW
    ngrp = seq_per_w // G
    mesh = plsc.VectorSubcoreMesh(core_axis_name="c", subcore_axis_name="s")

    @functools.partial(
        pl.kernel,
        out_type=jax.ShapeDtypeStruct((nb, S, D), jnp.float32),
        mesh=mesh,
        scratch_types=[
            pltpu.VMEM((seq_per_w, S), jnp.int32),
            pltpu.VMEM((G, S, D), jnp.float32),
            pltpu.SemaphoreType.DMA((G,)),
            pltpu.SemaphoreType.DMA,
        ],
        compiler_params=pltpu.CompilerParams(use_tc_tiling_on_sc=False),
    )
    def emb(tid_hbm, table_hbm, out_hbm, idx_v, rows_v, gsem, wsem):
        wid = lax.axis_index("s") * 2 + lax.axis_index("c")
        seq0 = wid * seq_per_w
        pltpu.sync_copy(tid_hbm.at[pl.ds(seq0, seq_per_w)], idx_v)

        def body(grp, _):
            r0 = grp * G
            gathers = [
                pltpu.async_copy(
                    table_hbm.at[idx_v.at[r0 + b]], rows_v.at[b], gsem.at[b]
                )
                for b in range(G)
            ]
            wbs = []
            for b in range(G):
                gathers[b].wait()
                wbs.append(
                    pltpu.async_copy(
                        rows_v.at[b], out_hbm.at[seq0 + r0 + b], wsem
                    )
                )
            for wb in wbs:
                wb.wait()
            return ()

        lax.fori_loop(0, ngrp, body, ())

    return emb


_emb_half = _make_kernel(B // 2)


@jax.jit
def kernel(token_ids, weight):
    out0 = _emb_half(token_ids[: B // 2], weight)
    out1 = _emb_half(token_ids[B // 2 :], weight)
    return jnp.concatenate([out0, out1], axis=0)
